# Initial kernel scaffold; baseline (speedup 1.0000x reference)
import jax
import jax.numpy as jnp
from jax.experimental import pallas as pl


def kernel(x, path, W, att, bias):
    raise NotImplementedError("write your pallas kernel here")


# speedup vs baseline: 86.4005x; 86.4005x over previous
import functools

import jax
import jax.numpy as jnp
from jax import lax
from jax.experimental import pallas as pl
from jax.experimental.pallas import tpu as pltpu
from jax.experimental.pallas import tpu_sc as plsc

N = 10000
E = 320000
EMB = 128
NH = 4
REPR = 16
HR = NH * REPR
NW = 32
EPW = E // NW
CH = 80
G = CH // 16
CHUNKS = EPW // CH
N_PAD = 10240
RPT = N_PAD // 16


def _prep_body(x_ref, w_ref, a_ref, h_ref, asat_ref):
    h = jnp.dot(x_ref[...], w_ref[...], preferred_element_type=jnp.float32)
    h_ref[...] = h
    asat_ref[...] = jnp.dot(h, a_ref[...], preferred_element_type=jnp.float32)


_prep = pl.pallas_call(
    _prep_body,
    out_shape=(
        jax.ShapeDtypeStruct((N, HR), jnp.float32),
        jax.ShapeDtypeStruct((N, 8), jnp.float32),
    ),
)


def _combine_body(o_ref, d_ref, b_ref, out_ref):
    o = o_ref[0] + o_ref[1]
    dsum = d_ref[0] + d_ref[1]
    parts = [jnp.broadcast_to(dsum[:, hh:hh + 1], (N_PAD, REPR)) for hh in range(NH)]
    d64 = jnp.concatenate(parts, axis=1)
    out_ref[...] = o / (d64 + 1e-16) + b_ref[...]


_combine = pl.pallas_call(
    _combine_body,
    out_shape=jax.ShapeDtypeStruct((N_PAD, HR), jnp.float32),
)

_mesh = plsc.VectorSubcoreMesh(core_axis_name="c", subcore_axis_name="s")


@functools.partial(
    pl.kernel,
    out_type=(
        jax.ShapeDtypeStruct((2, N_PAD, HR), jnp.float32),
        jax.ShapeDtypeStruct((2, N_PAD, 16), jnp.float32),
    ),
    mesh=_mesh,
    compiler_params=pltpu.CompilerParams(
        needs_layout_passes=False, use_tc_tiling_on_sc=False),
    scratch_types=[
        pltpu.VMEM((CH,), jnp.int32),
        pltpu.VMEM((CH,), jnp.int32),
        pltpu.VMEM((CH, HR), jnp.float32),
        pltpu.VMEM((CH, 16), jnp.float32),
        pltpu.VMEM((CH, 8), jnp.float32),
        pltpu.VMEM((CH, 8), jnp.float32),
        pltpu.VMEM_SHARED((N_PAD, HR), jnp.float32),
        pltpu.VMEM_SHARED((N_PAD, 16), jnp.float32),
        pltpu.SemaphoreType.DMA,
        pltpu.SemaphoreType.DMA,
        pltpu.SemaphoreType.DMA,
    ],
)
def _edge_kernel(asat_hbm, src_hbm, dst_hbm, h_hbm, out_raw, den_raw,
                 src_v, dst_v, rows_v, exr_v, as_v, at_v,
                 out_acc, den_acc, sem, sem2, sem3):
    c = lax.axis_index("c")
    s = lax.axis_index("s")
    wid = c * 16 + s
    r0 = s * RPT

    zero16 = jnp.zeros((16,), jnp.float32)

    def _zero_bufs(i, carry):
        exr_v[i, :] = zero16
        for k in range(NH):
            rows_v[i, pl.ds(k * 16, 16)] = zero16
        return carry

    lax.fori_loop(0, CH, _zero_bufs, 0)

    for k in range(RPT // CH):
        pltpu.sync_copy(rows_v, out_acc.at[pl.ds(r0 + k * CH, CH)])
        pltpu.sync_copy(exr_v, den_acc.at[pl.ds(r0 + k * CH, CH)])
    plsc.subcore_barrier()

    iot = lax.iota(jnp.int32, 16)

    def _chunk(ci, carry):
        base = wid * EPW + ci * CH
        pltpu.sync_copy(src_hbm.at[pl.ds(base, CH)], src_v)
        pltpu.sync_copy(dst_hbm.at[pl.ds(base, CH)], dst_v)
        cp1 = pltpu.async_copy(h_hbm.at[src_v], rows_v, sem)
        cp2 = pltpu.async_copy(asat_hbm.at[src_v], as_v, sem2)
        cp3 = pltpu.async_copy(asat_hbm.at[dst_v], at_v, sem3)
        cp2.wait()
        cp3.wait()
        for g in range(G):
            rowi = iot + (g * 16)
            for hh in range(NH):
                a_s = plsc.load_gather(
                    as_v, [rowi, jnp.full((16,), hh, jnp.int32)])
                a_t = plsc.load_gather(
                    at_v, [rowi, jnp.full((16,), 4 + hh, jnp.int32)])
                al = a_s + a_t
                al = jnp.maximum(al, al * 0.2)
                exh = jnp.exp(al)
                plsc.store_scatter(
                    exr_v, [rowi, jnp.full((16,), hh, jnp.int32)], exh)
        cp1.wait()

        def _scale(e, inner):
            exv = exr_v[e, :]
            for hh in range(NH):
                sc = exv[hh]
                rows_v[e, pl.ds(hh * 16, 16)] = rows_v[e, pl.ds(hh * 16, 16)] * sc
            return inner

        lax.fori_loop(0, CH, _scale, 0)
        pltpu.sync_copy(rows_v, out_acc.at[dst_v], add=True)
        pltpu.sync_copy(exr_v, den_acc.at[dst_v], add=True)
        return carry

    lax.fori_loop(0, CHUNKS, _chunk, 0)

    plsc.subcore_barrier()
    pltpu.sync_copy(out_acc.at[pl.ds(r0, RPT)], out_raw.at[c, pl.ds(r0, RPT)])
    pltpu.sync_copy(den_acc.at[pl.ds(r0, RPT)], den_raw.at[c, pl.ds(r0, RPT)])


def kernel(x, path, W, att, bias):
    att_r = att.reshape(NH, 2 * REPR)
    eye = jnp.eye(NH, dtype=jnp.float32)
    a_src = (att_r[:, :REPR, None] * eye[:, None, :]).reshape(HR, NH)
    a_dst = (att_r[:, REPR:, None] * eye[:, None, :]).reshape(HR, NH)
    A = jnp.concatenate([a_src, a_dst], axis=1)

    h, asat = _prep(x, W, A)
    out_raw, den_raw = _edge_kernel(asat, path[0], path[1], h)
    return _combine(out_raw, den_raw, bias.reshape(1, HR))[:N]
